# baseline (speedup 1.0000x reference)
import functools

import jax
import jax.numpy as jnp
from jax import lax
from jax.experimental import pallas as pl
from jax.experimental.pallas import tpu as pltpu
from jax.experimental.pallas import tpu_sc as plsc

VOCAB = 100000
D = 128
BATCH = 4096
SEQ = 200
N = BATCH * SEQ
NSEG = 3
NC, NS = 2, 16
NW = NC * NS
PER_W = N // NW
G = 128
STEPS = PER_W // G
SUBS = 8
PAIRS = SUBS // 2
NPAIR_BUF = 3
BODIES = STEPS // SUBS
LANES = 16
VBLK = 4000
VGRID = VOCAB // VBLK


def _build_body(tok_ref, seg_ref, out_ref):
    l = pl.program_id(0) % NSEG
    out_ref[...] = tok_ref[...] + seg_ref[pl.ds(l, 1), :]


@jax.jit
def _build_fused(token_table, seg8):
    return pl.pallas_call(
        _build_body,
        grid=(NSEG * VGRID,),
        in_specs=[
            pl.BlockSpec((VBLK, D), lambda i: (i // NSEG, 0)),
            pl.BlockSpec((8, D), lambda i: (0, 0)),
        ],
        out_specs=pl.BlockSpec((VBLK, D),
                               lambda i: ((i % NSEG) * VGRID + i // NSEG, 0)),
        out_shape=jax.ShapeDtypeStruct((NSEG * VOCAB, D), jnp.float32),
    )(token_table, seg8)


def _sc_body(seq_hbm, lab_hbm, fused_hbm, out_hbm,
             idx_v, lab_v, rows0, rows1, rows2,
             gsem0, gsem1, gsem2, gsem3,
             ssem0, ssem1, ssem2, ssem3):
    wid = lax.axis_index("s") * NC + lax.axis_index("c")
    base = wid * PER_W
    rows = (rows0, rows1, rows2)
    gsems = (gsem0, gsem1, gsem2, gsem3)
    ssems = (ssem0, ssem1, ssem2, ssem3)

    def loop_body(i, _):
        srow = pl.multiple_of(base // G + i * SUBS, 8)
        pltpu.sync_copy(seq_hbm.at[pl.ds(srow, SUBS)], idx_v)
        pltpu.sync_copy(lab_hbm.at[pl.ds(srow, SUBS)], lab_v)
        for k in range(SUBS):
            for g in range(G // LANES):
                sl = pl.ds(g * LANES, LANES)
                idx_v[k, sl] = idx_v[k, sl] + lab_v[k, sl] * VOCAB

        def fire(p, b):
            return [pltpu.async_copy(fused_hbm.at[idx_v.at[2 * p + j]],
                                     rows[b].at[pl.ds(j * G, G)], gsems[p])
                    for j in range(2)]

        gcp = [fire(p, p) for p in range(NPAIR_BUF)]
        scp = []
        for p in range(PAIRS):
            b = p % NPAIR_BUF
            if p == 2:
                scp[0].wait()
                gcp.append(fire(3, 0))
            for cp in gcp[p]:
                cp.wait()
            off = base + i * SUBS * G + p * 2 * G
            scp.append(pltpu.async_copy(
                rows[b], out_hbm.at[pl.ds(pl.multiple_of(off, 2 * G), 2 * G)],
                ssems[p]))
        for cp in scp[1:]:
            cp.wait()
        return 0

    lax.fori_loop(0, BODIES, loop_body, 0)


@jax.jit
def _embed(seq2d, lab2d, fused):
    fn = functools.partial(
        pl.kernel,
        out_type=jax.ShapeDtypeStruct((N, D), jnp.float32),
        mesh=plsc.VectorSubcoreMesh(core_axis_name="c", subcore_axis_name="s"),
        scratch_types=(
            [pltpu.VMEM((SUBS, G), jnp.int32),
             pltpu.VMEM((SUBS, G), jnp.int32)]
            + [pltpu.VMEM((2 * G, D), jnp.float32)] * NPAIR_BUF
            + [pltpu.SemaphoreType.DMA] * (2 * PAIRS)
        ),
    )(_sc_body)
    return fn(seq2d, lab2d, fused)


def kernel(sequence, segment_label, token_table, segment_table):
    seq2d = sequence.reshape(N // G, G)
    lab2d = segment_label.reshape(N // G, G)
    seg8 = jnp.pad(segment_table, ((0, 8 - NSEG), (0, 0)))
    fused = _build_fused(token_table, seg8)
    out = _embed(seq2d, lab2d, fused)
    return out.reshape(BATCH, SEQ, D)


# speedup vs baseline: 1.0497x; 1.0497x over previous
import functools

import jax
import jax.numpy as jnp
from jax import lax
from jax.experimental import pallas as pl
from jax.experimental.pallas import tpu as pltpu
from jax.experimental.pallas import tpu_sc as plsc

VOCAB = 100000
D = 128
BATCH = 4096
SEQ = 200
N = BATCH * SEQ
NSEG = 3
NC, NS = 2, 16
NW = NC * NS
PER_W = N // NW
G = 128
STEPS = PER_W // G
SUBS = 8
NBUF = 7
BODIES = STEPS // SUBS
LANES = 16
VBLK = 4000
VGRID = VOCAB // VBLK


def _build_body(tok_ref, seg_ref, out_ref):
    l = pl.program_id(0) % NSEG
    out_ref[...] = tok_ref[...] + seg_ref[pl.ds(l, 1), :]


@jax.jit
def _build_fused(token_table, seg8):
    return pl.pallas_call(
        _build_body,
        grid=(NSEG * VGRID,),
        in_specs=[
            pl.BlockSpec((VBLK, D), lambda i: (i // NSEG, 0)),
            pl.BlockSpec((8, D), lambda i: (0, 0)),
        ],
        out_specs=pl.BlockSpec((VBLK, D),
                               lambda i: ((i % NSEG) * VGRID + i // NSEG, 0)),
        out_shape=jax.ShapeDtypeStruct((NSEG * VOCAB, D), jnp.float32),
    )(token_table, seg8)


def _sc_body(seq_hbm, lab_hbm, fused_hbm, out_hbm,
             idx_v, lab_v,
             rows0, rows1, rows2, rows3, rows4, rows5, rows6,
             gsem0, gsem1, gsem2, gsem3, gsem4, gsem5, gsem6, gsem7,
             ssem0, ssem1, ssem2, ssem3, ssem4, ssem5, ssem6, ssem7):
    wid = lax.axis_index("s") * NC + lax.axis_index("c")
    base = wid * PER_W
    rows = (rows0, rows1, rows2, rows3, rows4, rows5, rows6)
    gsems = (gsem0, gsem1, gsem2, gsem3, gsem4, gsem5, gsem6, gsem7)
    ssems = (ssem0, ssem1, ssem2, ssem3, ssem4, ssem5, ssem6, ssem7)

    def stage(i):
        srow = pl.multiple_of(base // G + i * SUBS, 8)
        pltpu.sync_copy(seq_hbm.at[pl.ds(srow, SUBS)], idx_v)
        pltpu.sync_copy(lab_hbm.at[pl.ds(srow, SUBS)], lab_v)
        for k in range(SUBS):
            for g in range(G // LANES):
                sl = pl.ds(g * LANES, LANES)
                idx_v[k, sl] = idx_v[k, sl] + lab_v[k, sl] * VOCAB

    stage(0)

    def loop_body(i, _):
        gcp = [pltpu.async_copy(fused_hbm.at[idx_v.at[k]], rows[k], gsems[k])
               for k in range(NBUF)]
        scp = []
        for k in range(SUBS):
            b = k % NBUF
            if k == 4:
                scp[0].wait()
                gcp.append(pltpu.async_copy(fused_hbm.at[idx_v.at[NBUF]],
                                            rows[0], gsems[NBUF]))
            gcp[k].wait()
            off = base + (i * SUBS + k) * G
            scp.append(pltpu.async_copy(
                rows[b], out_hbm.at[pl.ds(pl.multiple_of(off, G), G)],
                ssems[k]))

        @pl.when(i + 1 < BODIES)
        def _():
            stage(i + 1)

        for cp in scp[1:]:
            cp.wait()
        return 0

    lax.fori_loop(0, BODIES, loop_body, 0)


@jax.jit
def _embed(seq2d, lab2d, fused):
    fn = functools.partial(
        pl.kernel,
        out_type=jax.ShapeDtypeStruct((N, D), jnp.float32),
        mesh=plsc.VectorSubcoreMesh(core_axis_name="c", subcore_axis_name="s"),
        scratch_types=(
            [pltpu.VMEM((SUBS, G), jnp.int32),
             pltpu.VMEM((SUBS, G), jnp.int32)]
            + [pltpu.VMEM((G, D), jnp.float32)] * NBUF
            + [pltpu.SemaphoreType.DMA] * (2 * SUBS)
        ),
    )(_sc_body)
    return fn(seq2d, lab2d, fused)


def kernel(sequence, segment_label, token_table, segment_table):
    seq2d = sequence.reshape(N // G, G)
    lab2d = segment_label.reshape(N // G, G)
    seg8 = jnp.pad(segment_table, ((0, 8 - NSEG), (0, 0)))
    fused = _build_fused(token_table, seg8)
    out = _embed(seq2d, lab2d, fused)
    return out.reshape(BATCH, SEQ, D)


# speedup vs baseline: 1.0899x; 1.0383x over previous
import functools

import jax
import jax.numpy as jnp
from jax import lax
from jax.experimental import pallas as pl
from jax.experimental.pallas import tpu as pltpu
from jax.experimental.pallas import tpu_sc as plsc

VOCAB = 100000
D = 128
BATCH = 4096
SEQ = 200
N = BATCH * SEQ
NSEG = 3
NC, NS = 2, 16
NW = NC * NS
PER_W = N // NW
G = 128
STEPS = PER_W // G
SUBS = 8
BODIES = STEPS // SUBS
NBUF = 7
LOOKAHEAD = 5
VBLK = 4000
VGRID = VOCAB // VBLK
FROWS = N // G


def _build_body(tok_ref, seg_ref, out_ref):
    l = pl.program_id(0) % NSEG
    out_ref[...] = tok_ref[...] + seg_ref[pl.ds(l, 1), :]


@jax.jit
def _build_fused(token_table, seg8):
    return pl.pallas_call(
        _build_body,
        grid=(NSEG * VGRID,),
        in_specs=[
            pl.BlockSpec((VBLK, D), lambda i: (i // NSEG, 0)),
            pl.BlockSpec((8, D), lambda i: (0, 0)),
        ],
        out_specs=pl.BlockSpec((VBLK, D),
                               lambda i: ((i % NSEG) * VGRID + i // NSEG, 0)),
        out_shape=jax.ShapeDtypeStruct((NSEG * VOCAB, D), jnp.float32),
    )(token_table, seg8)


def _fidx_body(seq_ref, lab_ref, out_ref):
    out_ref[...] = lab_ref[...] * VOCAB + seq_ref[...]


@jax.jit
def _build_fidx(seq2d, lab2d):
    blk = FROWS // 8
    return pl.pallas_call(
        _fidx_body,
        grid=(8,),
        in_specs=[pl.BlockSpec((blk, G), lambda i: (i, 0)),
                  pl.BlockSpec((blk, G), lambda i: (i, 0))],
        out_specs=pl.BlockSpec((blk, G), lambda i: (i, 0)),
        out_shape=jax.ShapeDtypeStruct((FROWS, G), jnp.int32),
    )(seq2d, lab2d)


def _sc_body(fidx_hbm, fused_hbm, out_hbm,
             idx_a, idx_b, rows0, rows1, rows2, rows3, rows4, rows5, rows6,
             gsem0, gsem1, gsem2, gsem3, gsem4, gsem5, gsem6, gsem7,
             ssem0, ssem1, ssem2, ssem3, ssem4, ssem5, ssem6, ssem7):
    wid = lax.axis_index("s") * NC + lax.axis_index("c")
    baserow = wid * BODIES * SUBS
    base = wid * PER_W
    idxb = (idx_a, idx_b)
    rows = (rows0, rows1, rows2, rows3, rows4, rows5, rows6)
    gsems = (gsem0, gsem1, gsem2, gsem3, gsem4, gsem5, gsem6, gsem7)
    ssems = (ssem0, ssem1, ssem2, ssem3, ssem4, ssem5, ssem6, ssem7)

    def stage(j):
        srow = pl.multiple_of(baserow + j * SUBS, 8)
        pltpu.sync_copy(fidx_hbm.at[pl.ds(srow, SUBS)], idxb[j % 2])

    def fire_gather(s):
        j = s // SUBS
        return pltpu.async_copy(fused_hbm.at[idxb[j % 2].at[s % SUBS]],
                                rows[s % NBUF], gsems[s % 8])

    def fire_store(s):
        off = pl.multiple_of(base + s * G, G)
        return pltpu.async_copy(rows[s % NBUF], out_hbm.at[pl.ds(off, G)],
                                ssems[s % 8])

    stage(0)
    gcp = {}
    scp = {}
    for s in range(LOOKAHEAD):
        gcp[s] = fire_gather(s)
    for s in range(STEPS):
        nxt = s + LOOKAHEAD
        if nxt % SUBS == 0 and nxt // SUBS < BODIES:
            stage(nxt // SUBS)
        if s >= 2:
            scp[s - 2].wait()
        if nxt < STEPS:
            gcp[nxt] = fire_gather(nxt)
        gcp[s].wait()
        scp[s] = fire_store(s)
    scp[STEPS - 2].wait()
    scp[STEPS - 1].wait()


@jax.jit
def _embed(fidx, fused):
    fn = functools.partial(
        pl.kernel,
        out_type=jax.ShapeDtypeStruct((N, D), jnp.float32),
        mesh=plsc.VectorSubcoreMesh(core_axis_name="c", subcore_axis_name="s"),
        scratch_types=(
            [pltpu.VMEM((SUBS, G), jnp.int32)] * 2
            + [pltpu.VMEM((G, D), jnp.float32)] * NBUF
            + [pltpu.SemaphoreType.DMA] * 16
        ),
    )(_sc_body)
    return fn(fidx, fused)


def kernel(sequence, segment_label, token_table, segment_table):
    seq2d = sequence.reshape(FROWS, G)
    lab2d = segment_label.reshape(FROWS, G)
    seg8 = jnp.pad(segment_table, ((0, 8 - NSEG), (0, 0)))
    fused = _build_fused(token_table, seg8)
    fidx = _build_fidx(seq2d, lab2d)
    out = _embed(fidx, fused)
    return out.reshape(BATCH, SEQ, D)


# speedup vs baseline: 1.0918x; 1.0017x over previous
import functools

import jax
import jax.numpy as jnp
from jax import lax
from jax.experimental import pallas as pl
from jax.experimental.pallas import tpu as pltpu
from jax.experimental.pallas import tpu_sc as plsc

VOCAB = 100000
D = 128
BATCH = 4096
SEQ = 200
N = BATCH * SEQ
NSEG = 3
NC, NS = 2, 16
NW = NC * NS
PER_W = N // NW
G = 128
STEPS = PER_W // G
SUBS = 8
BODIES = STEPS // SUBS
NBUF = 7
LOOKAHEAD = 5
VBLK = 4000
VGRID = VOCAB // VBLK
FROWS = N // G


def _build_body(tok_ref, seg_ref, out_ref):
    l = pl.program_id(0) % NSEG
    out_ref[...] = tok_ref[...] + seg_ref[pl.ds(l, 1), :]


@jax.jit
def _build_fused(token_table, seg8):
    return pl.pallas_call(
        _build_body,
        grid=(NSEG * VGRID,),
        in_specs=[
            pl.BlockSpec((VBLK, D), lambda i: (i // NSEG, 0)),
            pl.BlockSpec((8, D), lambda i: (0, 0)),
        ],
        out_specs=pl.BlockSpec((VBLK, D),
                               lambda i: ((i % NSEG) * VGRID + i // NSEG, 0)),
        out_shape=jax.ShapeDtypeStruct((NSEG * VOCAB, D), jnp.float32),
    )(token_table, seg8)


def _fidx_body(seq_ref, lab_ref, out_ref):
    out_ref[...] = lab_ref[...] * VOCAB + seq_ref[...]


@jax.jit
def _build_fidx(seq2d, lab2d):
    blk = FROWS // 8
    return pl.pallas_call(
        _fidx_body,
        grid=(8,),
        in_specs=[pl.BlockSpec((blk, G), lambda i: (i, 0)),
                  pl.BlockSpec((blk, G), lambda i: (i, 0))],
        out_specs=pl.BlockSpec((blk, G), lambda i: (i, 0)),
        out_shape=jax.ShapeDtypeStruct((FROWS, G), jnp.int32),
    )(seq2d, lab2d)


def _sc_body(fidx_hbm, fused_hbm, out_hbm,
             idx_a, idx_b, rows0, rows1, rows2, rows3, rows4, rows5, rows6,
             gsem0, gsem1, gsem2, gsem3, gsem4, gsem5, gsem6, gsem7,
             ssem0, ssem1, ssem2, ssem3, ssem4, ssem5, ssem6, ssem7,
             isem0, isem1):
    wid = lax.axis_index("s") * NC + lax.axis_index("c")
    baserow = wid * BODIES * SUBS
    base = wid * PER_W
    idxb = (idx_a, idx_b)
    isems = (isem0, isem1)
    rows = (rows0, rows1, rows2, rows3, rows4, rows5, rows6)
    gsems = (gsem0, gsem1, gsem2, gsem3, gsem4, gsem5, gsem6, gsem7)
    ssems = (ssem0, ssem1, ssem2, ssem3, ssem4, ssem5, ssem6, ssem7)

    def stage(j):
        srow = pl.multiple_of(baserow + j * SUBS, 8)
        return pltpu.async_copy(fidx_hbm.at[pl.ds(srow, SUBS)], idxb[j % 2],
                                isems[j % 2])

    def fire_gather(s):
        j = s // SUBS
        return pltpu.async_copy(fused_hbm.at[idxb[j % 2].at[s % SUBS]],
                                rows[s % NBUF], gsems[s % 8])

    def fire_store(s):
        off = pl.multiple_of(base + s * G, G)
        return pltpu.async_copy(rows[s % NBUF], out_hbm.at[pl.ds(off, G)],
                                ssems[s % 8])

    gcp = {}
    scp = {}
    stcp = {0: stage(0)}
    stcp[0].wait()
    if BODIES > 1:
        stcp[1] = stage(1)
    for s in range(LOOKAHEAD):
        gcp[s] = fire_gather(s)
    for s in range(STEPS):
        nxt = s + LOOKAHEAD
        if s >= 2:
            scp[s - 2].wait()
        if nxt < STEPS:
            if nxt % SUBS == 0:
                stcp[nxt // SUBS].wait()
            gcp[nxt] = fire_gather(nxt)
        gcp[s].wait()
        if (s + 1) % SUBS == 0 and (s + 1) // SUBS + 1 < BODIES:
            j1 = (s + 1) // SUBS + 1
            stcp[j1] = stage(j1)
        scp[s] = fire_store(s)
    scp[STEPS - 2].wait()
    scp[STEPS - 1].wait()


@jax.jit
def _embed(fidx, fused):
    fn = functools.partial(
        pl.kernel,
        out_type=jax.ShapeDtypeStruct((N, D), jnp.float32),
        mesh=plsc.VectorSubcoreMesh(core_axis_name="c", subcore_axis_name="s"),
        scratch_types=(
            [pltpu.VMEM((SUBS, G), jnp.int32)] * 2
            + [pltpu.VMEM((G, D), jnp.float32)] * NBUF
            + [pltpu.SemaphoreType.DMA] * 18
        ),
    )(_sc_body)
    return fn(fidx, fused)


def kernel(sequence, segment_label, token_table, segment_table):
    seq2d = sequence.reshape(FROWS, G)
    lab2d = segment_label.reshape(FROWS, G)
    seg8 = jnp.pad(segment_table, ((0, 8 - NSEG), (0, 0)))
    fused = _build_fused(token_table, seg8)
    fidx = _build_fidx(seq2d, lab2d)
    out = _embed(fidx, fused)
    return out.reshape(BATCH, SEQ, D)
